# baseline (speedup 1.0000x reference)
import functools

import jax
import jax.numpy as jnp
from jax import lax
from jax.experimental import pallas as pl
from jax.experimental.pallas import tpu as pltpu
from jax.experimental.pallas import tpu_sc as plsc

_TOPK = 2
_E = 8
_BLK = 512
_NB = 4096 // _BLK + _E
_PADT = _NB * _BLK


def _glu_body(be_ref, bv_ref, xs_ref, wg_ref, wu_ref, wd_ref, ys_ref):
    b = pl.program_id(0)
    F = wg_ref.shape[2]
    FH = F // 2

    @pl.when(bv_ref[b] == 1)
    def _():
        xb = xs_ref[...]
        for i in range(2):
            fs = pl.ds(i * FH, FH)
            h = jnp.dot(xb, wg_ref[0, :, fs], preferred_element_type=jnp.float32)
            u = jnp.dot(xb, wu_ref[0, :, fs], preferred_element_type=jnp.float32)
            act = h * jax.lax.logistic(h) * u
            yp = jnp.dot(act, wd_ref[0, fs, :], preferred_element_type=jnp.float32)
            if i == 0:
                ys_ref[...] = yp
            else:
                ys_ref[...] += yp

    @pl.when(bv_ref[b] == 0)
    def _():
        ys_ref[...] = jnp.zeros_like(ys_ref)


def _make_meta(T):
    info = plsc.get_sparse_core_info()
    NC, NS, L = info.num_cores, info.num_subcores, info.num_lanes
    NW = NC * NS
    per_w = T // NW
    n_g = per_w // L
    mesh = plsc.VectorSubcoreMesh(core_axis_name="c", subcore_axis_name="s")
    dn = lax.GatherDimensionNumbers(
        offset_dims=(), collapsed_slice_dims=(0,), start_index_map=(0,))
    pib = lax.GatherScatterMode.PROMISE_IN_BOUNDS

    @functools.partial(
        pl.kernel, mesh=mesh,
        out_type=(
            jax.ShapeDtypeStruct((NW, L), jnp.int32),
            jax.ShapeDtypeStruct((T,), jnp.int32),
            jax.ShapeDtypeStruct((T,), jnp.int32),
        ),
        scratch_types=[
            pltpu.VMEM((per_w,), jnp.int32),
            pltpu.VMEM((per_w,), jnp.int32),
            pltpu.VMEM((per_w,), jnp.int32),
            pltpu.VMEM((per_w,), jnp.int32),
            pltpu.VMEM((L,), jnp.int32),
        ],
    )
    def meta(e0_hbm, e1_hbm, lc_hbm, r0_hbm, r1_hbm,
             e0v, e1v, r0v, r1v, lcv):
        wid = lax.axis_index("s") * NC + lax.axis_index("c")
        base = wid * per_w
        pltpu.sync_copy(e0_hbm.at[pl.ds(base, per_w)], e0v)
        pltpu.sync_copy(e1_hbm.at[pl.ds(base, per_w)], e1v)
        lanes = lax.iota(jnp.int32, L)
        lc = jnp.zeros((L,), jnp.int32)
        for g in range(n_g):
            sl = pl.ds(g * L, L)
            for ev_ref, rv_ref in ((e0v, r0v), (e1v, r1v)):
                ev = ev_ref[sl]
                rank = jnp.zeros((L,), jnp.int32)
                for e in range(_E):
                    m = ev == e
                    cs = jnp.where(m, 1, 0).astype(jnp.int32)
                    for sh in (1, 2, 4, 8):
                        idx = jnp.maximum(lanes - sh, 0)[:, None]
                        shifted = lax.gather(cs, idx, dn, (1,), mode=pib)
                        cs = cs + jnp.where(lanes >= sh, shifted, 0)
                    lastidx = jnp.full((L, 1), L - 1, jnp.int32)
                    totv = lax.gather(cs, lastidx, dn, (1,), mode=pib)
                    eidx = jnp.full((L, 1), e, jnp.int32)
                    lce = lax.gather(lc, eidx, dn, (1,), mode=pib)
                    rank = jnp.where(m, lce + cs - 1, rank)
                    lc = lc + jnp.where(lanes == e, totv, 0)
                rv_ref[sl] = rank
        lcv[...] = lc
        pltpu.sync_copy(r0v, r0_hbm.at[pl.ds(base, per_w)])
        pltpu.sync_copy(r1v, r1_hbm.at[pl.ds(base, per_w)])
        pltpu.sync_copy(lcv, lc_hbm.at[wid])

    return meta


def _make_dispatch(T, D, PADT):
    info = plsc.get_sparse_core_info()
    NC, NS, L = info.num_cores, info.num_subcores, info.num_lanes
    NW = NC * NS
    per_w = T // NW
    mesh = plsc.VectorSubcoreMesh(core_axis_name="c", subcore_axis_name="s")

    @functools.partial(
        pl.kernel, mesh=mesh,
        out_type=jax.ShapeDtypeStruct((PADT, D), jnp.float32),
        scratch_types=[
            pltpu.VMEM((per_w,), jnp.int32),
            pltpu.VMEM((per_w,), jnp.int32),
            pltpu.VMEM((per_w, D), jnp.float32),
            pltpu.SemaphoreType.DMA,
        ],
    )
    def disp(x_hbm, i0_hbm, i1_hbm, xs_hbm, i0v, i1v, xv, sem):
        wid = lax.axis_index("s") * NC + lax.axis_index("c")
        base = wid * per_w
        pltpu.sync_copy(i0_hbm.at[pl.ds(base, per_w)], i0v)
        pltpu.sync_copy(i1_hbm.at[pl.ds(base, per_w)], i1v)
        pltpu.sync_copy(x_hbm.at[pl.ds(base, per_w)], xv)
        c0 = pltpu.async_copy(xv, xs_hbm.at[i0v], sem)
        c1 = pltpu.async_copy(xv, xs_hbm.at[i1v], sem)
        c0.wait()
        c1.wait()

    return disp


def _make_combine(T, D, PADT):
    info = plsc.get_sparse_core_info()
    NC, NS, L = info.num_cores, info.num_subcores, info.num_lanes
    NW = NC * NS
    per_w = T // NW
    n_ch = per_w // L
    mesh = plsc.VectorSubcoreMesh(core_axis_name="c", subcore_axis_name="s")

    dn = lax.GatherDimensionNumbers(
        offset_dims=(), collapsed_slice_dims=(0,), start_index_map=(0,))
    pib = lax.GatherScatterMode.PROMISE_IN_BOUNDS

    @functools.partial(
        pl.kernel, mesh=mesh,
        out_type=jax.ShapeDtypeStruct((T, D), jnp.float32),
        scratch_types=[
            pltpu.VMEM((per_w,), jnp.int32),
            pltpu.VMEM((per_w,), jnp.int32),
            pltpu.VMEM((per_w,), jnp.float32),
            pltpu.VMEM((per_w,), jnp.float32),
            pltpu.VMEM((L, D), jnp.float32),
            pltpu.VMEM((L, D), jnp.float32),
            pltpu.VMEM((L, D), jnp.float32),
            pltpu.VMEM((L, D), jnp.float32),
            pltpu.VMEM((L, D), jnp.float32),
            pltpu.VMEM((L, D), jnp.float32),
            pltpu.SemaphoreType.DMA,
            pltpu.SemaphoreType.DMA,
        ],
    )
    def comb(ys_hbm, i0_hbm, i1_hbm, a0_hbm, a1_hbm, out_hbm,
             i0v, i1v, a0v, a1v, r0a, r0b, r1a, r1b, oba, obb, semg, sems):
        wid = lax.axis_index("s") * NC + lax.axis_index("c")
        base = wid * per_w
        pltpu.sync_copy(i0_hbm.at[pl.ds(base, per_w)], i0v)
        pltpu.sync_copy(i1_hbm.at[pl.ds(base, per_w)], i1v)
        pltpu.sync_copy(a0_hbm.at[pl.ds(base, per_w)], a0v)
        pltpu.sync_copy(a1_hbm.at[pl.ds(base, per_w)], a1v)
        r0 = [r0a, r0b]
        r1 = [r1a, r1b]
        ob = [oba, obb]

        def fire(c):
            p = c & 1
            g0 = pltpu.async_copy(ys_hbm.at[i0v.at[pl.ds(c * L, L)]], r0[p], semg)
            g1 = pltpu.async_copy(ys_hbm.at[i1v.at[pl.ds(c * L, L)]], r1[p], semg)
            return g0, g1

        pend = fire(0)
        stores = [None, None]
        for c in range(n_ch):
            p = c & 1
            pend[0].wait()
            pend[1].wait()
            if c + 1 < n_ch:
                pend = fire(c + 1)
            if stores[p] is not None:
                stores[p].wait()
            a0g = a0v[pl.ds(c * L, L)]
            a1g = a1v[pl.ds(c * L, L)]
            rp0, rp1, obp = r0[p], r1[p], ob[p]

            def tok(t, carry, a0g=a0g, a1g=a1g, rp0=rp0, rp1=rp1, obp=obp):
                lane = jnp.full((L, 1), t, jnp.int32)
                a0 = lax.gather(a0g, lane, dn, (1,), mode=pib)
                a1 = lax.gather(a1g, lane, dn, (1,), mode=pib)
                for j in range(D // L):
                    sl = pl.ds(j * L, L)
                    obp[t, sl] = a0 * rp0[t, sl] + a1 * rp1[t, sl]
                return carry

            lax.fori_loop(0, L, tok, 0)
            stores[p] = pltpu.async_copy(
                obp, out_hbm.at[pl.ds(base + c * L, L)], sems)
        for st in stores:
            if st is not None:
                st.wait()

    return comb


def kernel(x, router_w, e_score_bias, w_gate, w_up, w_down):
    T, D = x.shape
    F = w_gate.shape[2]

    logits = jnp.dot(x, router_w.T)
    scores = jax.nn.sigmoid(logits.astype(jnp.float32))
    scores_for_choice = scores + e_score_bias[None, :]
    _, topk_idx = jax.lax.top_k(scores_for_choice, _TOPK)
    topk_scores = jnp.take_along_axis(scores, topk_idx, axis=1)
    aff = topk_scores / (jnp.sum(topk_scores, axis=1, keepdims=True) + 1e-9)

    e0 = topk_idx[:, 0].astype(jnp.int32)
    e1 = topk_idx[:, 1].astype(jnp.int32)
    meta = _make_meta(T)
    lc, r0, r1 = meta(e0, e1)
    colcum = jnp.cumsum(lc, axis=0)
    wexcl = colcum - lc
    counts = colcum[-1]
    padded = ((counts + _BLK - 1) // _BLK) * _BLK
    cum_padded = jnp.cumsum(padded)
    blk_off = cum_padded - padded
    base_we = (wexcl + blk_off[None, :]).reshape(-1)
    per_w = T // 32
    w_of_t = (jnp.arange(T, dtype=jnp.int32) // per_w) * 16
    i0 = base_we[w_of_t + e0] + r0
    i1 = base_we[w_of_t + e1] + r1

    starts = jnp.arange(_NB, dtype=jnp.int32) * _BLK
    cpadE = cum_padded[:_E]
    be_raw = jnp.sum((starts[:, None] >= cpadE[None, :]).astype(jnp.int32), axis=1)
    valid = (starts < cpadE[-1]).astype(jnp.int32)
    nvalid = jnp.sum(valid)
    last_e = be_raw[nvalid - 1]
    block_expert = jnp.where(valid == 1, be_raw, last_e).astype(jnp.int32)

    disp = _make_dispatch(T, D, _PADT)
    xs = disp(x, i0, i1)

    grid_spec = pltpu.PrefetchScalarGridSpec(
        num_scalar_prefetch=2,
        grid=(_NB,),
        in_specs=[
            pl.BlockSpec((_BLK, D), lambda b, be, bv: (b, 0)),
            pl.BlockSpec((1, D, F), lambda b, be, bv: (be[b], 0, 0)),
            pl.BlockSpec((1, D, F), lambda b, be, bv: (be[b], 0, 0)),
            pl.BlockSpec((1, F, D), lambda b, be, bv: (be[b], 0, 0)),
        ],
        out_specs=pl.BlockSpec((_BLK, D), lambda b, be, bv: (b, 0)),
    )
    ys = pl.pallas_call(
        _glu_body,
        grid_spec=grid_spec,
        out_shape=jax.ShapeDtypeStruct((_PADT, D), jnp.float32),
        compiler_params=pltpu.CompilerParams(
            dimension_semantics=("arbitrary",),
            vmem_limit_bytes=128 * 1024 * 1024,
        ),
    )(block_expert, valid, xs, w_gate, w_up, w_down)

    comb = _make_combine(T, D, _PADT)
    out = comb(ys, i0, i1, aff[:, 0], aff[:, 1])
    return out.astype(x.dtype)


# speedup vs baseline: 1.1475x; 1.1475x over previous
import functools

import jax
import jax.numpy as jnp
from jax import lax
from jax.experimental import pallas as pl
from jax.experimental.pallas import tpu as pltpu
from jax.experimental.pallas import tpu_sc as plsc

_TOPK = 2
_E = 8
_BLK = 512
_NB = 4096 // _BLK + _E
_PADT = _NB * _BLK


def _glu_body(be_ref, bv_ref, xs_ref, wg_ref, wu_ref, wd_ref, ys_ref):
    b = pl.program_id(0)
    F = wg_ref.shape[2]
    FH = F // 2

    @pl.when(bv_ref[b] == 1)
    def _():
        xb = xs_ref[...]
        for i in range(2):
            fs = pl.ds(i * FH, FH)
            h = jnp.dot(xb, wg_ref[0, :, fs], preferred_element_type=jnp.float32)
            u = jnp.dot(xb, wu_ref[0, :, fs], preferred_element_type=jnp.float32)
            act = h * jax.lax.logistic(h) * u
            yp = jnp.dot(act, wd_ref[0, fs, :], preferred_element_type=jnp.float32)
            if i == 0:
                ys_ref[...] = yp
            else:
                ys_ref[...] += yp

    @pl.when(bv_ref[b] == 0)
    def _():
        ys_ref[...] = jnp.zeros_like(ys_ref)


def _make_dispatch(T, D, PADT):
    info = plsc.get_sparse_core_info()
    NC, NS, L = info.num_cores, info.num_subcores, info.num_lanes
    NW = NC * NS
    per_w = T // NW
    mesh = plsc.VectorSubcoreMesh(core_axis_name="c", subcore_axis_name="s")

    @functools.partial(
        pl.kernel, mesh=mesh,
        out_type=jax.ShapeDtypeStruct((PADT, D), jnp.float32),
        scratch_types=[
            pltpu.VMEM((per_w,), jnp.int32),
            pltpu.VMEM((per_w,), jnp.int32),
            pltpu.VMEM((per_w, D), jnp.float32),
            pltpu.SemaphoreType.DMA,
        ],
    )
    def disp(x_hbm, i0_hbm, i1_hbm, xs_hbm, i0v, i1v, xv, sem):
        wid = lax.axis_index("s") * NC + lax.axis_index("c")
        base = wid * per_w
        pltpu.sync_copy(i0_hbm.at[pl.ds(base, per_w)], i0v)
        pltpu.sync_copy(i1_hbm.at[pl.ds(base, per_w)], i1v)
        pltpu.sync_copy(x_hbm.at[pl.ds(base, per_w)], xv)
        c0 = pltpu.async_copy(xv, xs_hbm.at[i0v], sem)
        c1 = pltpu.async_copy(xv, xs_hbm.at[i1v], sem)
        c0.wait()
        c1.wait()

    return disp


def _make_combine(T, D, PADT):
    info = plsc.get_sparse_core_info()
    NC, NS, L = info.num_cores, info.num_subcores, info.num_lanes
    NW = NC * NS
    per_w = T // NW
    n_ch = per_w // L
    mesh = plsc.VectorSubcoreMesh(core_axis_name="c", subcore_axis_name="s")

    dn = lax.GatherDimensionNumbers(
        offset_dims=(), collapsed_slice_dims=(0,), start_index_map=(0,))
    pib = lax.GatherScatterMode.PROMISE_IN_BOUNDS

    @functools.partial(
        pl.kernel, mesh=mesh,
        out_type=jax.ShapeDtypeStruct((T, D), jnp.float32),
        scratch_types=[
            pltpu.VMEM((per_w,), jnp.int32),
            pltpu.VMEM((per_w,), jnp.int32),
            pltpu.VMEM((per_w,), jnp.float32),
            pltpu.VMEM((per_w,), jnp.float32),
            pltpu.VMEM((L, D), jnp.float32),
            pltpu.VMEM((L, D), jnp.float32),
            pltpu.VMEM((L, D), jnp.float32),
            pltpu.VMEM((L, D), jnp.float32),
            pltpu.VMEM((L, D), jnp.float32),
            pltpu.VMEM((L, D), jnp.float32),
            pltpu.SemaphoreType.DMA,
            pltpu.SemaphoreType.DMA,
        ],
    )
    def comb(ys_hbm, i0_hbm, i1_hbm, a0_hbm, a1_hbm, out_hbm,
             i0v, i1v, a0v, a1v, r0a, r0b, r1a, r1b, oba, obb, semg, sems):
        wid = lax.axis_index("s") * NC + lax.axis_index("c")
        base = wid * per_w
        pltpu.sync_copy(i0_hbm.at[pl.ds(base, per_w)], i0v)
        pltpu.sync_copy(i1_hbm.at[pl.ds(base, per_w)], i1v)
        pltpu.sync_copy(a0_hbm.at[pl.ds(base, per_w)], a0v)
        pltpu.sync_copy(a1_hbm.at[pl.ds(base, per_w)], a1v)
        r0 = [r0a, r0b]
        r1 = [r1a, r1b]
        ob = [oba, obb]

        def fire(c):
            p = c & 1
            g0 = pltpu.async_copy(ys_hbm.at[i0v.at[pl.ds(c * L, L)]], r0[p], semg)
            g1 = pltpu.async_copy(ys_hbm.at[i1v.at[pl.ds(c * L, L)]], r1[p], semg)
            return g0, g1

        pend = fire(0)
        stores = [None, None]
        for c in range(n_ch):
            p = c & 1
            pend[0].wait()
            pend[1].wait()
            if c + 1 < n_ch:
                pend = fire(c + 1)
            if stores[p] is not None:
                stores[p].wait()
            a0g = a0v[pl.ds(c * L, L)]
            a1g = a1v[pl.ds(c * L, L)]
            rp0, rp1, obp = r0[p], r1[p], ob[p]

            def tok(t, carry, a0g=a0g, a1g=a1g, rp0=rp0, rp1=rp1, obp=obp):
                lane = jnp.full((L, 1), t, jnp.int32)
                a0 = lax.gather(a0g, lane, dn, (1,), mode=pib)
                a1 = lax.gather(a1g, lane, dn, (1,), mode=pib)
                for j in range(D // L):
                    sl = pl.ds(j * L, L)
                    obp[t, sl] = a0 * rp0[t, sl] + a1 * rp1[t, sl]
                return carry

            lax.fori_loop(0, L, tok, 0)
            stores[p] = pltpu.async_copy(
                obp, out_hbm.at[pl.ds(base + c * L, L)], sems)
        for st in stores:
            if st is not None:
                st.wait()

    return comb


def kernel(x, router_w, e_score_bias, w_gate, w_up, w_down):
    T, D = x.shape
    F = w_gate.shape[2]

    logits = jnp.dot(x, router_w.T)
    scores = jax.nn.sigmoid(logits.astype(jnp.float32))
    scores_for_choice = scores + e_score_bias[None, :]
    _, topk_idx = jax.lax.top_k(scores_for_choice, _TOPK)
    topk_scores = jnp.take_along_axis(scores, topk_idx, axis=1)
    aff = topk_scores / (jnp.sum(topk_scores, axis=1, keepdims=True) + 1e-9)

    e_flat = topk_idx.reshape(-1).astype(jnp.int32)
    oh = (e_flat[:, None] == jnp.arange(_E, dtype=jnp.int32)[None, :]).astype(jnp.int32)
    incl = jnp.cumsum(oh, axis=0)
    ranks = jnp.take_along_axis(incl, e_flat[:, None], axis=1)[:, 0] - 1
    counts = incl[-1]
    padded = ((counts + _BLK - 1) // _BLK) * _BLK
    cum_padded = jnp.cumsum(padded)
    blk_off = cum_padded - padded
    pos = blk_off[e_flat] + ranks
    pos2 = pos.reshape(T, _TOPK)
    i0 = pos2[:, 0]
    i1 = pos2[:, 1]

    starts = jnp.arange(_NB, dtype=jnp.int32) * _BLK
    be_raw = jnp.sum((starts[:, None] >= cum_padded[None, :]).astype(jnp.int32), axis=1)
    valid = (starts < cum_padded[-1]).astype(jnp.int32)
    nvalid = jnp.sum(valid)
    last_e = be_raw[nvalid - 1]
    block_expert = jnp.where(valid == 1, be_raw, last_e).astype(jnp.int32)

    disp = _make_dispatch(T, D, _PADT)
    xs = disp(x, i0, i1)

    grid_spec = pltpu.PrefetchScalarGridSpec(
        num_scalar_prefetch=2,
        grid=(_NB,),
        in_specs=[
            pl.BlockSpec((_BLK, D), lambda b, be, bv: (b, 0)),
            pl.BlockSpec((1, D, F), lambda b, be, bv: (be[b], 0, 0)),
            pl.BlockSpec((1, D, F), lambda b, be, bv: (be[b], 0, 0)),
            pl.BlockSpec((1, F, D), lambda b, be, bv: (be[b], 0, 0)),
        ],
        out_specs=pl.BlockSpec((_BLK, D), lambda b, be, bv: (b, 0)),
    )
    ys = pl.pallas_call(
        _glu_body,
        grid_spec=grid_spec,
        out_shape=jax.ShapeDtypeStruct((_PADT, D), jnp.float32),
        compiler_params=pltpu.CompilerParams(
            dimension_semantics=("arbitrary",),
            vmem_limit_bytes=128 * 1024 * 1024,
        ),
    )(block_expert, valid, xs, w_gate, w_up, w_down)

    comb = _make_combine(T, D, _PADT)
    out = comb(ys, i0, i1, aff[:, 0], aff[:, 1])
    return out.astype(x.dtype)
